# baseline (speedup 1.0000x reference)
import functools

import jax
import jax.numpy as jnp
from jax import lax
from jax.experimental import pallas as pl
from jax.experimental.pallas import tpu as pltpu
from jax.experimental.pallas import tpu_sc as plsc

NUM_NODES = 100000
H_DIM = 64
BATCH = 16384
OUT_W = 128

_NC = 2
_NS = 16
_NW = _NC * _NS
_B_PER_W = BATCH // _NW
_C = 4
_CH = _B_PER_W // _C
_IDX_ROWS = BATCH // 128


def _make_squeeze():
    mesh = plsc.VectorSubcoreMesh(core_axis_name="c", subcore_axis_name="s")

    @functools.partial(
        pl.kernel,
        mesh=mesh,
        compiler_params=pltpu.CompilerParams(needs_layout_passes=False),
        out_type=jax.ShapeDtypeStruct((_IDX_ROWS, 128), jnp.int32),
        scratch_types=[
            pltpu.VMEM((_B_PER_W, 1), jnp.int32),
            pltpu.VMEM((_C, 128), jnp.int32),
        ],
    )
    def squeeze_kernel(h_hbm, out_hbm, col_v, idx_v):
        wid = lax.axis_index("s") * _NC + lax.axis_index("c")
        base = wid * _B_PER_W
        pltpu.sync_copy(h_hbm.at[pl.ds(base, _B_PER_W)], col_v)
        zeros16 = jnp.zeros((16,), jnp.int32)
        for r in range(_C):
            for j in range(8):
                rows = lax.iota(jnp.int32, 16) + (r * 128 + j * 16)
                idx_v[r, pl.ds(j * 16, 16)] = plsc.load_gather(
                    col_v, [rows, zeros16]
                )
        pltpu.sync_copy(idx_v, out_hbm.at[pl.ds(wid * _C, _C)])

    return squeeze_kernel


def _make_gather():
    mesh = plsc.VectorSubcoreMesh(core_axis_name="c", subcore_axis_name="s")

    @functools.partial(
        pl.kernel,
        mesh=mesh,
        compiler_params=pltpu.CompilerParams(
            use_tc_tiling_on_sc=False, needs_layout_passes=False
        ),
        out_type=jax.ShapeDtypeStruct((BATCH, OUT_W), jnp.float32),
        scratch_types=[
            pltpu.VMEM((_C, 128), jnp.int32),
            pltpu.VMEM((_C, _CH, H_DIM), jnp.float32),
            pltpu.SemaphoreType.DMA,
            pltpu.SemaphoreType.DMA,
        ],
    )
    def gather_kernel(idx_hbm, table_hbm, out_hbm, idx_v, rows_v,
                      gsem, wsem):
        wid = lax.axis_index("s") * _NC + lax.axis_index("c")
        base = wid * _B_PER_W
        pltpu.sync_copy(idx_hbm.at[pl.ds(wid * _C, _C)], idx_v)
        gathers = [
            pltpu.async_copy(
                table_hbm.at[idx_v.at[c]],
                rows_v.at[c],
                gsem,
            )
            for c in range(_C)
        ]
        writebacks = []
        for c in range(_C):
            gathers[c].wait()
            writebacks.append(
                pltpu.async_copy(
                    rows_v.at[c],
                    out_hbm.at[pl.ds(base + c * _CH, _CH), pl.ds(0, H_DIM)],
                    wsem,
                )
            )
        for wb in writebacks:
            wb.wait()

    return gather_kernel


_squeeze = _make_squeeze()
_gather = _make_gather()


def kernel(g, h, table):
    idx = _squeeze(h)
    out_padded = _gather(idx, table)
    return out_padded[:, :H_DIM]


# speedup vs baseline: 1.0806x; 1.0806x over previous
import functools

import jax
import jax.numpy as jnp
from jax import lax
from jax.experimental import pallas as pl
from jax.experimental.pallas import tpu as pltpu
from jax.experimental.pallas import tpu_sc as plsc

NUM_NODES = 100000
H_DIM = 64
BATCH = 16384
W = 128

_NC = 2
_NS = 16
_NW = _NC * _NS
_B_PER_W = BATCH // _NW
_C = 4
_CH = _B_PER_W // _C
_IDX_ROWS = BATCH // 128


def _make_gather():
    mesh = plsc.VectorSubcoreMesh(core_axis_name="c", subcore_axis_name="s")

    @functools.partial(
        pl.kernel,
        mesh=mesh,
        compiler_params=pltpu.CompilerParams(
            use_tc_tiling_on_sc=False, needs_layout_passes=False
        ),
        out_type=jax.ShapeDtypeStruct((BATCH, W), jnp.float32),
        scratch_types=[
            pltpu.VMEM((_C, 128), jnp.int32),
            pltpu.VMEM((_C, _CH, W), jnp.float32),
            pltpu.SemaphoreType.DMA,
            pltpu.SemaphoreType.DMA,
        ],
    )
    def gather_kernel(idx_hbm, table_hbm, out_hbm, idx_v, rows_v,
                      gsem, wsem):
        wid = lax.axis_index("s") * _NC + lax.axis_index("c")
        base = wid * _B_PER_W
        pltpu.sync_copy(idx_hbm.at[pl.ds(wid * _C, _C)], idx_v)
        gathers = [
            pltpu.async_copy(
                table_hbm.at[idx_v.at[c]],
                rows_v.at[c],
                gsem,
            )
            for c in range(_C)
        ]
        writebacks = []
        for c in range(_C):
            gathers[c].wait()
            writebacks.append(
                pltpu.async_copy(
                    rows_v.at[c],
                    out_hbm.at[pl.ds(base + c * _CH, _CH)],
                    wsem,
                )
            )
        for wb in writebacks:
            wb.wait()

    return gather_kernel


_gather = _make_gather()


def kernel(g, h, table):
    idx128 = h.reshape(_IDX_ROWS, 128)
    table128 = jnp.pad(table, ((0, 0), (0, W - H_DIM)))
    out_padded = _gather(idx128, table128)
    return out_padded[:, :H_DIM]
